# Initial kernel scaffold; baseline (speedup 1.0000x reference)
import jax
import jax.numpy as jnp
from jax.experimental import pallas as pl


def kernel(pl_x, pl_edge_index, pl_edge_attr, p_x, p_edge_index, p_edge_attr, l_x, l_edge_index, l_edge_attr, pl_Wl, pl_bl, pl_Wr, p_Wl, p_bl, p_Wr, l_Wl, l_bl, l_Wr):
    raise NotImplementedError("write your pallas kernel here")


# speedup vs baseline: 4.3547x; 4.3547x over previous
import functools

import jax
import jax.numpy as jnp
from jax import lax
from jax.experimental import pallas as pl
from jax.experimental.pallas import tpu as pltpu
from jax.experimental.pallas import tpu_sc as plsc

HIDDEN = 128
N_BIG = 10000
N_PAD = 10240
E_BIG = 320000
NCONF = 4
NC = 2
NS = 16
CHUNK = 80
EPT = E_BIG // NS
NCHUNK = EPT // CHUNK
RPT = N_PAD // NS
TEMPERATURE = 298.0
RT = 1.98720425864083 / 1000 * TEMPERATURE
F32 = jnp.float32


def _fill_const_buffers(zrow, zflat, ones_v=None):
    if ones_v is not None:
        for i in range(CHUNK // 16):
            ones_v[pl.ds(i * 16, 16)] = jnp.ones((16,), F32)

    def zr(i, carry):
        for j in range(HIDDEN // 16):
            zrow[i, pl.ds(j * 16, 16)] = jnp.zeros((16,), F32)
        return carry

    lax.fori_loop(0, zrow.shape[0], zr, 0)

    def zf(i, carry):
        zflat[pl.ds(i * 16, 16)] = jnp.zeros((16,), F32)
        return carry

    lax.fori_loop(0, zflat.shape[0] // 16, zf, 0)


def _zero_shared(s, acc, vec_acc, zrow, zflat):
    rbase = s * RPT
    nfull = RPT // 128
    for t in range(nfull):
        pltpu.sync_copy(zrow, acc.at[pl.ds(rbase + t * 128, 128)])
    tail = RPT - nfull * 128
    if tail:
        pltpu.sync_copy(zrow.at[pl.ds(0, tail)],
                        acc.at[pl.ds(rbase + nfull * 128, tail)])
    zc = zflat.shape[0]
    nvc = N_PAD // zc

    @pl.when(s < nvc)
    def _():
        pltpu.sync_copy(zflat, vec_acc.at[pl.ds(s * zc, zc)])


def _adjust_idx(src_buf, dst_buf, noff):
    for k in range(CHUNK // 16):
        dst_buf[pl.ds(k * 16, 16)] = src_buf[pl.ds(k * 16, 16)] + noff


def _sc_first_body(src_hbm, dst_hbm, x_hbm, agg_hbm, d_hbm,
                   sidx, didx, sadj, rows, ones_v, zrow, zflat,
                   acc, dacc, sem):
    c = lax.axis_index("c")
    s = lax.axis_index("s")
    _fill_const_buffers(zrow, zflat, ones_v)
    for cg in range(NCONF // NC):
        conf = c + NC * cg
        noff = conf * N_PAD
        _zero_shared(s, acc, dacc, zrow, zflat)
        plsc.subcore_barrier()
        ebase = conf * E_BIG + s * EPT

        def chunk(g, carry):
            off = ebase + g * CHUNK
            pltpu.sync_copy(src_hbm.at[pl.ds(off, CHUNK)], sidx)
            pltpu.sync_copy(dst_hbm.at[pl.ds(off, CHUNK)], didx)
            _adjust_idx(sidx, sadj, noff)
            pltpu.async_copy(x_hbm.at[sadj], rows, sem).wait()
            pltpu.sync_copy(rows, acc.at[didx], add=True)
            pltpu.sync_copy(ones_v, dacc.at[didx], add=True)
            return carry

        lax.fori_loop(0, NCHUNK, chunk, 0)
        plsc.subcore_barrier()
        rbase = s * RPT
        pltpu.sync_copy(acc.at[pl.ds(rbase, RPT)],
                        agg_hbm.at[pl.ds(noff + rbase, RPT)])

        @pl.when(s == 0)
        def _():
            pltpu.sync_copy(dacc, d_hbm.at[pl.ds(noff, N_PAD)])

        plsc.subcore_barrier()


def _sc_second_body(src_hbm, dst_hbm, h_hbm, w_hbm, agg_hbm, c_hbm,
                    sidx, didx, sadj, dadj, rows, wv, zrow, zflat,
                    acc, cacc, sem, sem2):
    c = lax.axis_index("c")
    s = lax.axis_index("s")
    _fill_const_buffers(zrow, zflat)
    for cg in range(NCONF // NC):
        conf = c + NC * cg
        noff = conf * N_PAD
        _zero_shared(s, acc, cacc, zrow, zflat)
        plsc.subcore_barrier()
        ebase = conf * E_BIG + s * EPT

        def chunk(g, carry):
            off = ebase + g * CHUNK
            pltpu.sync_copy(src_hbm.at[pl.ds(off, CHUNK)], sidx)
            pltpu.sync_copy(dst_hbm.at[pl.ds(off, CHUNK)], didx)
            _adjust_idx(sidx, sadj, noff)
            _adjust_idx(didx, dadj, noff)
            pltpu.async_copy(h_hbm.at[sadj], rows, sem).wait()
            pltpu.sync_copy(rows, acc.at[didx], add=True)
            pltpu.async_copy(w_hbm.at[dadj], wv, sem2).wait()
            pltpu.sync_copy(wv, cacc.at[sidx], add=True)
            return carry

        lax.fori_loop(0, NCHUNK, chunk, 0)
        plsc.subcore_barrier()
        rbase = s * RPT
        pltpu.sync_copy(acc.at[pl.ds(rbase, RPT)],
                        agg_hbm.at[pl.ds(noff + rbase, RPT)])

        @pl.when(s == 0)
        def _():
            pltpu.sync_copy(cacc, c_hbm.at[pl.ds(noff, N_PAD)])

        plsc.subcore_barrier()


def _sc_aggregate_first(src_all, dst_all, x_all):
    mesh = plsc.VectorSubcoreMesh(core_axis_name="c", subcore_axis_name="s")
    return pl.kernel(
        _sc_first_body,
        mesh=mesh,
        out_type=[
            jax.ShapeDtypeStruct((NCONF * N_PAD, HIDDEN), F32),
            jax.ShapeDtypeStruct((NCONF * N_PAD,), F32),
        ],
        scratch_types=[
            pltpu.VMEM((CHUNK,), jnp.int32),
            pltpu.VMEM((CHUNK,), jnp.int32),
            pltpu.VMEM((CHUNK,), jnp.int32),
            pltpu.VMEM((CHUNK, HIDDEN), F32),
            pltpu.VMEM((CHUNK,), F32),
            pltpu.VMEM((128, HIDDEN), F32),
            pltpu.VMEM((2048,), F32),
            pltpu.VMEM_SHARED((N_PAD, HIDDEN), F32),
            pltpu.VMEM_SHARED((N_PAD,), F32),
            pltpu.SemaphoreType.DMA,
        ],
    )(src_all, dst_all, x_all)


def _sc_aggregate_second(src_all, dst_all, h_all, w_all):
    mesh = plsc.VectorSubcoreMesh(core_axis_name="c", subcore_axis_name="s")
    return pl.kernel(
        _sc_second_body,
        mesh=mesh,
        out_type=[
            jax.ShapeDtypeStruct((NCONF * N_PAD, HIDDEN), F32),
            jax.ShapeDtypeStruct((NCONF * N_PAD,), F32),
        ],
        scratch_types=[
            pltpu.VMEM((CHUNK,), jnp.int32),
            pltpu.VMEM((CHUNK,), jnp.int32),
            pltpu.VMEM((CHUNK,), jnp.int32),
            pltpu.VMEM((CHUNK,), jnp.int32),
            pltpu.VMEM((CHUNK, HIDDEN), F32),
            pltpu.VMEM((CHUNK,), F32),
            pltpu.VMEM((128, HIDDEN), F32),
            pltpu.VMEM((2048,), F32),
            pltpu.VMEM_SHARED((N_PAD, HIDDEN), F32),
            pltpu.VMEM_SHARED((N_PAD,), F32),
            pltpu.SemaphoreType.DMA,
            pltpu.SemaphoreType.DMA,
        ],
    )(src_all, dst_all, h_all, w_all)


ROWB = 2048
NBLK = NCONF * N_PAD // ROWB


def _dotT(a, b):
    return lax.dot_general(a, b, (((1,), (1,)), ((), ())),
                           preferred_element_type=F32)


def _tc_layer1_body(agg_ref, d_ref, x_ref, wl_ref, bl_ref, wr_ref,
                    h1_ref, w_ref):
    w = 1.0 / jnp.maximum(d_ref[...], 1.0)
    w_ref[...] = w
    mean = agg_ref[...] * w
    h = _dotT(mean, wl_ref[0, 0]) + bl_ref[0, 0][None, :] + _dotT(x_ref[...], wr_ref[0, 0])
    h1_ref[...] = jnp.maximum(h, 0.0)


def _tc_layer1(agg1, d_col, x_all, wl_s, bl_s, wr_s):
    return pl.pallas_call(
        _tc_layer1_body,
        grid=(NBLK,),
        in_specs=[
            pl.BlockSpec((ROWB, HIDDEN), lambda i: (i, 0)),
            pl.BlockSpec((ROWB, 1), lambda i: (i, 0)),
            pl.BlockSpec((ROWB, HIDDEN), lambda i: (i, 0)),
            pl.BlockSpec((1, 3, HIDDEN, HIDDEN), lambda i: (i // (NBLK // 2), 0, 0, 0)),
            pl.BlockSpec((1, 3, HIDDEN), lambda i: (i // (NBLK // 2), 0, 0)),
            pl.BlockSpec((1, 3, HIDDEN, HIDDEN), lambda i: (i // (NBLK // 2), 0, 0, 0)),
        ],
        out_specs=[
            pl.BlockSpec((ROWB, HIDDEN), lambda i: (i, 0)),
            pl.BlockSpec((ROWB, 1), lambda i: (i, 0)),
        ],
        out_shape=[
            jax.ShapeDtypeStruct((NCONF * N_PAD, HIDDEN), F32),
            jax.ShapeDtypeStruct((NCONF * N_PAD, 1), F32),
        ],
    )(agg1, d_col, x_all, wl_s, bl_s, wr_s)


def _tc_layer2_body(agg_ref, h1_ref, w_ref, c_ref, wl_ref, bl_ref, wr_ref,
                    es_ref):
    i = pl.program_id(0)
    blocks_per_conf = NBLK // NCONF
    mean = agg_ref[...] * w_ref[...]
    h2 = _dotT(mean, wl_ref[0, 1]) + bl_ref[0, 1][None, :] + _dotT(h1_ref[...], wr_ref[0, 1])
    h2 = jnp.maximum(h2, 0.0)
    s_l = jnp.sum(wl_ref[0, 2], axis=0)[:, None]
    s_r = jnp.sum(wr_ref[0, 2], axis=0)[:, None]
    t = lax.dot_general(h2, s_l, (((1,), (0,)), ((), ())),
                        preferred_element_type=F32)
    u = lax.dot_general(h2, s_r, (((1,), (0,)), ((), ())),
                        preferred_element_type=F32)
    row0 = (i % blocks_per_conf) * ROWB
    node_id = row0 + lax.broadcasted_iota(jnp.int32, (ROWB, 1), 0)
    valid = node_id < N_BIG
    contrib = jnp.where(valid, c_ref[...] * t + u, 0.0)

    @pl.when(i % blocks_per_conf == 0)
    def _():
        es_ref[...] = (N_BIG * jnp.sum(bl_ref[0, 2])).reshape(1, 1, 1)

    es_ref[...] += jnp.sum(contrib).reshape(1, 1, 1)


def _tc_layer2(agg2, h1, w_col, c_col, wl_s, bl_s, wr_s):
    return pl.pallas_call(
        _tc_layer2_body,
        grid=(NBLK,),
        in_specs=[
            pl.BlockSpec((ROWB, HIDDEN), lambda i: (i, 0)),
            pl.BlockSpec((ROWB, HIDDEN), lambda i: (i, 0)),
            pl.BlockSpec((ROWB, 1), lambda i: (i, 0)),
            pl.BlockSpec((ROWB, 1), lambda i: (i, 0)),
            pl.BlockSpec((1, 3, HIDDEN, HIDDEN), lambda i: (i // (NBLK // 2), 0, 0, 0)),
            pl.BlockSpec((1, 3, HIDDEN), lambda i: (i // (NBLK // 2), 0, 0)),
            pl.BlockSpec((1, 3, HIDDEN, HIDDEN), lambda i: (i // (NBLK // 2), 0, 0, 0)),
        ],
        out_specs=pl.BlockSpec((1, 1, 1), lambda i: (i // (NBLK // NCONF), 0, 0)),
        out_shape=jax.ShapeDtypeStruct((NCONF, 1, 1), F32),
    )(agg2, h1, w_col, c_col, wl_s, bl_s, wr_s)


L_N = 64
L_E = 2048
L_G = 20


def _tc_ligand_body(x_ref, src_ref, dst_ref, wl_ref, bl_ref, wr_ref, out_ref):
    src = src_ref[0, 0, :]
    dst = dst_ref[0, 0, :]
    iota = lax.broadcasted_iota(jnp.int32, (L_E, L_N), 1)
    oh_s = (src[:, None] == iota).astype(F32)
    oh_d = (dst[:, None] == iota).astype(F32)
    A = lax.dot_general(oh_d, oh_s, (((0,), (0,)), ((), ())),
                        preferred_element_type=F32)
    denom = jnp.maximum(jnp.sum(A, axis=1, keepdims=True), 1.0)
    h = x_ref[0]
    for i in range(3):
        agg = lax.dot_general(A, h, (((1,), (0,)), ((), ())),
                              preferred_element_type=F32)
        h = _dotT(agg / denom, wl_ref[i]) + bl_ref[i][None, :] + _dotT(h, wr_ref[i])
        if i < 2:
            h = jnp.maximum(h, 0.0)
    out_ref[...] = jnp.sum(h).reshape(1, 1, 1)


def _tc_ligand(l_x, l_src, l_dst, l_Wl, l_bl, l_Wr):
    return pl.pallas_call(
        _tc_ligand_body,
        grid=(L_G,),
        in_specs=[
            pl.BlockSpec((1, L_N, HIDDEN), lambda i: (i, 0, 0)),
            pl.BlockSpec((1, 1, L_E), lambda i: (i, 0, 0)),
            pl.BlockSpec((1, 1, L_E), lambda i: (i, 0, 0)),
            pl.BlockSpec((3, HIDDEN, HIDDEN), lambda i: (0, 0, 0)),
            pl.BlockSpec((3, HIDDEN), lambda i: (0, 0)),
            pl.BlockSpec((3, HIDDEN, HIDDEN), lambda i: (0, 0, 0)),
        ],
        out_specs=pl.BlockSpec((1, 1, 1), lambda i: (i, 0, 0)),
        out_shape=jax.ShapeDtypeStruct((L_G, 1, 1), F32),
    )(l_x, l_src, l_dst, l_Wl, l_bl, l_Wr)


def kernel(pl_x, pl_edge_index, pl_edge_attr, p_x, p_edge_index, p_edge_attr,
           l_x, l_edge_index, l_edge_attr,
           pl_Wl, pl_bl, pl_Wr, p_Wl, p_bl, p_Wr, l_Wl, l_bl, l_Wr):
    del pl_edge_attr, p_edge_attr, l_edge_attr
    x_all = jnp.pad(jnp.concatenate([pl_x, p_x]),
                    ((0, 0), (0, N_PAD - N_BIG), (0, 0))).reshape(NCONF * N_PAD, HIDDEN)
    src_all = jnp.concatenate(
        [pl_edge_index[:, 0, :], p_edge_index[:, 0, :]]).astype(jnp.int32).reshape(-1)
    dst_all = jnp.concatenate(
        [pl_edge_index[:, 1, :], p_edge_index[:, 1, :]]).astype(jnp.int32).reshape(-1)
    wl_s = jnp.stack([pl_Wl, p_Wl])
    bl_s = jnp.stack([pl_bl, p_bl])
    wr_s = jnp.stack([pl_Wr, p_Wr])

    agg1, d = _sc_aggregate_first(src_all, dst_all, x_all)
    h1, w_col = _tc_layer1(agg1, d.reshape(-1, 1), x_all, wl_s, bl_s, wr_s)
    agg2, cvec = _sc_aggregate_second(src_all, dst_all, h1, w_col.reshape(-1))
    es = _tc_layer2(agg2, h1, w_col, cvec.reshape(-1, 1), wl_s, bl_s, wr_s)

    l_src = l_edge_index[:, 0:1, :].astype(jnp.int32)
    l_dst = l_edge_index[:, 1:2, :].astype(jnp.int32)
    l_es = _tc_ligand(l_x, l_src, l_dst, l_Wl, l_bl, l_Wr)

    pl_avg = jnp.mean(es[0:2, 0, 0])
    p_avg = jnp.mean(es[2:4, 0, 0])
    l_avg = jnp.mean(l_es[:, 0, 0])
    return (pl_avg - p_avg - l_avg) / (-RT)


# speedup vs baseline: 7.7214x; 1.7731x over previous
import jax
import jax.numpy as jnp
from jax import lax
from jax.experimental import pallas as pl
from jax.experimental.pallas import tpu as pltpu
from jax.experimental.pallas import tpu_sc as plsc

HIDDEN = 128
N_BIG = 10000
N_PAD = 10240
E_BIG = 320000
NCONF = 4
NC = 2
NS = 16
CHUNK = 80
KB = 2
EPT = E_BIG // NS
NSUPER = EPT // (KB * CHUNK)
NPAIR = (NSUPER + 1) // 2
RPT = N_PAD // NS
ZROWS = 16
TEMPERATURE = 298.0
RT = 1.98720425864083 / 1000 * TEMPERATURE
F32 = jnp.float32


def _fill_zrow_zvec(zrow, zvec):
    def zr(i, carry):
        for j in range(HIDDEN // 16):
            zrow[i, pl.ds(j * 16, 16)] = jnp.zeros((16,), F32)
        return carry

    lax.fori_loop(0, zrow.shape[0], zr, 0)

    def zv(i, carry):
        zvec[pl.ds(i * 16, 16)] = jnp.zeros((16,), F32)
        return carry

    lax.fori_loop(0, zvec.shape[0] // 16, zv, 0)


def _zero_slices(s, acc, vec_acc, zrow, zvec):
    rbase = s * RPT
    for t in range(RPT // ZROWS):
        pltpu.sync_copy(zrow, acc.at[pl.ds(rbase + t * ZROWS, ZROWS)])
    pltpu.sync_copy(zvec, vec_acc.at[pl.ds(rbase, RPT)])


def _load_adj(srcb, dstb, r, sidx, didx, sadj, noff):
    pltpu.sync_copy(srcb.at[r], sidx)
    pltpu.sync_copy(dstb.at[r], didx)
    for j in range(KB):
        for k in range(CHUNK // 16):
            sl = pl.ds(k * 16, 16)
            sadj[j, sl] = sidx[j, sl] + noff


def _fire_gathers(tbl, sadj, rows, sem):
    for j in range(KB):
        pltpu.async_copy(tbl.at[sadj.at[j]], rows.at[j], sem)


def _drain_gathers(tbl, sadj, rows, sem):
    for j in range(KB):
        pltpu.make_async_copy(tbl.at[sadj.at[j]], rows.at[j], sem).wait()


def _fire_scatters(rows, acc, didx, sem):
    for j in range(KB):
        pltpu.async_copy(rows.at[j], acc.at[didx.at[j]], sem, add=True)


def _drain_scatters(rows, acc, didx, sem):
    for j in range(KB):
        pltpu.make_async_copy(rows.at[j], acc.at[didx.at[j]], sem).wait()


def _fire_vec_scatters(vals, vacc, idx, sem):
    for j in range(KB):
        pltpu.async_copy(vals.at[j], vacc.at[idx.at[j]], sem, add=True)


def _drain_vec_scatters(vals, vacc, idx, sem):
    for j in range(KB):
        pltpu.make_async_copy(vals.at[j], vacc.at[idx.at[j]], sem).wait()


def _sc_first_body(srcb, dstb, x_hbm, agg_hbm, w_hbm,
                   sidx0, didx0, sadj0, sidx1, didx1, sadj1,
                   rows0, rows1, ones2, wtmp, zrow, zvec,
                   acc, dacc, g0s, g1s, s0s, s1s, o0s, o1s):
    c = lax.axis_index("c")
    s = lax.axis_index("s")
    _fill_zrow_zvec(zrow, zvec)
    for j in range(KB):
        for i in range(CHUNK // 16):
            ones2[j, pl.ds(i * 16, 16)] = jnp.ones((16,), F32)
    for cg in range(NCONF // NC):
        conf = c + NC * cg
        noff = conf * N_PAD
        _zero_slices(s, acc, dacc, zrow, zvec)
        plsc.subcore_barrier()
        base3 = (conf * NS + s) * NSUPER

        _load_adj(srcb, dstb, base3, sidx0, didx0, sadj0, noff)
        _fire_gathers(x_hbm, sadj0, rows0, g0s)

        def it(i, carry):
            sc1 = 2 * i + 1
            sc2 = 2 * i + 2
            _drain_gathers(x_hbm, sadj0, rows0, g0s)
            _fire_scatters(rows0, acc, didx0, s0s)
            _fire_vec_scatters(ones2, dacc, didx0, o0s)

            @pl.when(i > 0)
            def _():
                _drain_scatters(rows1, acc, didx1, s1s)
                _drain_vec_scatters(ones2, dacc, didx1, o1s)

            @pl.when(sc1 < NSUPER)
            def _():
                _load_adj(srcb, dstb, base3 + sc1, sidx1, didx1, sadj1, noff)
                _fire_gathers(x_hbm, sadj1, rows1, g1s)
                _drain_gathers(x_hbm, sadj1, rows1, g1s)
                _fire_scatters(rows1, acc, didx1, s1s)
                _fire_vec_scatters(ones2, dacc, didx1, o1s)

            _drain_scatters(rows0, acc, didx0, s0s)
            _drain_vec_scatters(ones2, dacc, didx0, o0s)

            @pl.when(sc2 < NSUPER)
            def _():
                _load_adj(srcb, dstb, base3 + sc2, sidx0, didx0, sadj0, noff)
                _fire_gathers(x_hbm, sadj0, rows0, g0s)

            return carry

        lax.fori_loop(0, NPAIR, it, 0)
        plsc.subcore_barrier()

        rb = s * RPT
        pltpu.sync_copy(dacc.at[pl.ds(rb, RPT)], wtmp)

        def winv(i, carry):
            sl = pl.ds(i * 16, 16)
            wtmp[sl] = 1.0 / jnp.maximum(wtmp[sl], 1.0)
            return carry

        lax.fori_loop(0, RPT // 16, winv, 0)
        pltpu.sync_copy(wtmp, w_hbm.at[pl.ds(noff + rb, RPT)])
        pltpu.sync_copy(acc.at[pl.ds(rb, RPT)], agg_hbm.at[pl.ds(noff + rb, RPT)])
        plsc.subcore_barrier()


def _sc_second_body(srcb, dstb, h_hbm, w_hbm, agg_hbm, c_hbm,
                    sidx0, didx0, sadj0, dadj0, sidx1, didx1, sadj1, dadj1,
                    rows0, rows1, wv0, wv1, zrow, zvec,
                    acc, cacc, g0s, g1s, s0s, s1s, o0s, o1s):
    c = lax.axis_index("c")
    s = lax.axis_index("s")
    _fill_zrow_zvec(zrow, zvec)

    def _load_adj2(r, sidx, didx, sadj, dadj, noff):
        pltpu.sync_copy(srcb.at[r], sidx)
        pltpu.sync_copy(dstb.at[r], didx)
        for j in range(KB):
            for k in range(CHUNK // 16):
                sl = pl.ds(k * 16, 16)
                sadj[j, sl] = sidx[j, sl] + noff
                dadj[j, sl] = didx[j, sl] + noff

    for cg in range(NCONF // NC):
        conf = c + NC * cg
        noff = conf * N_PAD
        _zero_slices(s, acc, cacc, zrow, zvec)
        plsc.subcore_barrier()
        base3 = (conf * NS + s) * NSUPER

        _load_adj2(base3, sidx0, didx0, sadj0, dadj0, noff)
        _fire_gathers(h_hbm, sadj0, rows0, g0s)
        _fire_gathers(w_hbm, dadj0, wv0, g0s)

        def it(i, carry):
            sc1 = 2 * i + 1
            sc2 = 2 * i + 2
            _drain_gathers(h_hbm, sadj0, rows0, g0s)
            _drain_gathers(w_hbm, dadj0, wv0, g0s)
            _fire_scatters(rows0, acc, didx0, s0s)
            _fire_vec_scatters(wv0, cacc, sidx0, o0s)

            @pl.when(i > 0)
            def _():
                _drain_scatters(rows1, acc, didx1, s1s)
                _drain_vec_scatters(wv1, cacc, sidx1, o1s)

            @pl.when(sc1 < NSUPER)
            def _():
                _load_adj2(base3 + sc1, sidx1, didx1, sadj1, dadj1, noff)
                _fire_gathers(h_hbm, sadj1, rows1, g1s)
                _fire_gathers(w_hbm, dadj1, wv1, g1s)
                _drain_gathers(h_hbm, sadj1, rows1, g1s)
                _drain_gathers(w_hbm, dadj1, wv1, g1s)
                _fire_scatters(rows1, acc, didx1, s1s)
                _fire_vec_scatters(wv1, cacc, sidx1, o1s)

            _drain_scatters(rows0, acc, didx0, s0s)
            _drain_vec_scatters(wv0, cacc, sidx0, o0s)

            @pl.when(sc2 < NSUPER)
            def _():
                _load_adj2(base3 + sc2, sidx0, didx0, sadj0, dadj0, noff)
                _fire_gathers(h_hbm, sadj0, rows0, g0s)
                _fire_gathers(w_hbm, dadj0, wv0, g0s)

            return carry

        lax.fori_loop(0, NPAIR, it, 0)
        plsc.subcore_barrier()
        rb = s * RPT
        pltpu.sync_copy(acc.at[pl.ds(rb, RPT)], agg_hbm.at[pl.ds(noff + rb, RPT)])
        pltpu.sync_copy(cacc.at[pl.ds(rb, RPT)], c_hbm.at[pl.ds(noff + rb, RPT)])
        plsc.subcore_barrier()


def _sc_aggregate_first(srcb, dstb, x_all):
    mesh = plsc.VectorSubcoreMesh(core_axis_name="c", subcore_axis_name="s")
    return pl.kernel(
        _sc_first_body,
        mesh=mesh,
        out_type=[
            jax.ShapeDtypeStruct((NCONF * N_PAD, HIDDEN), F32),
            jax.ShapeDtypeStruct((NCONF * N_PAD,), F32),
        ],
        scratch_types=[
            pltpu.VMEM((KB, CHUNK), jnp.int32),
            pltpu.VMEM((KB, CHUNK), jnp.int32),
            pltpu.VMEM((KB, CHUNK), jnp.int32),
            pltpu.VMEM((KB, CHUNK), jnp.int32),
            pltpu.VMEM((KB, CHUNK), jnp.int32),
            pltpu.VMEM((KB, CHUNK), jnp.int32),
            pltpu.VMEM((KB, CHUNK, HIDDEN), F32),
            pltpu.VMEM((KB, CHUNK, HIDDEN), F32),
            pltpu.VMEM((KB, CHUNK), F32),
            pltpu.VMEM((RPT,), F32),
            pltpu.VMEM((ZROWS, HIDDEN), F32),
            pltpu.VMEM((RPT,), F32),
            pltpu.VMEM_SHARED((N_PAD, HIDDEN), F32),
            pltpu.VMEM_SHARED((N_PAD,), F32),
            pltpu.SemaphoreType.DMA,
            pltpu.SemaphoreType.DMA,
            pltpu.SemaphoreType.DMA,
            pltpu.SemaphoreType.DMA,
            pltpu.SemaphoreType.DMA,
            pltpu.SemaphoreType.DMA,
        ],
    )(srcb, dstb, x_all)


def _sc_aggregate_second(srcb, dstb, h_all, w_flat):
    mesh = plsc.VectorSubcoreMesh(core_axis_name="c", subcore_axis_name="s")
    return pl.kernel(
        _sc_second_body,
        mesh=mesh,
        out_type=[
            jax.ShapeDtypeStruct((NCONF * N_PAD, HIDDEN), F32),
            jax.ShapeDtypeStruct((NCONF * N_PAD,), F32),
        ],
        scratch_types=[
            pltpu.VMEM((KB, CHUNK), jnp.int32),
            pltpu.VMEM((KB, CHUNK), jnp.int32),
            pltpu.VMEM((KB, CHUNK), jnp.int32),
            pltpu.VMEM((KB, CHUNK), jnp.int32),
            pltpu.VMEM((KB, CHUNK), jnp.int32),
            pltpu.VMEM((KB, CHUNK), jnp.int32),
            pltpu.VMEM((KB, CHUNK), jnp.int32),
            pltpu.VMEM((KB, CHUNK), jnp.int32),
            pltpu.VMEM((KB, CHUNK, HIDDEN), F32),
            pltpu.VMEM((KB, CHUNK, HIDDEN), F32),
            pltpu.VMEM((KB, CHUNK), F32),
            pltpu.VMEM((KB, CHUNK), F32),
            pltpu.VMEM((ZROWS, HIDDEN), F32),
            pltpu.VMEM((RPT,), F32),
            pltpu.VMEM_SHARED((N_PAD, HIDDEN), F32),
            pltpu.VMEM_SHARED((N_PAD,), F32),
            pltpu.SemaphoreType.DMA,
            pltpu.SemaphoreType.DMA,
            pltpu.SemaphoreType.DMA,
            pltpu.SemaphoreType.DMA,
            pltpu.SemaphoreType.DMA,
            pltpu.SemaphoreType.DMA,
        ],
    )(srcb, dstb, h_all, w_flat)


ROWB = 2048
NBLK = NCONF * N_PAD // ROWB


def _dotT(a, b):
    return lax.dot_general(a, b, (((1,), (1,)), ((), ())),
                           preferred_element_type=F32,
                           precision=lax.Precision.HIGHEST)


def _tc_layer1_body(agg_ref, w_ref, x_ref, wl_ref, bl_ref, wr_ref, h1_ref):
    mean = agg_ref[...] * w_ref[...]
    h = _dotT(mean, wl_ref[0, 0]) + bl_ref[0, 0][None, :] + _dotT(x_ref[...], wr_ref[0, 0])
    h1_ref[...] = jnp.maximum(h, 0.0)


def _tc_layer1(agg1, w_col, x_all, wl_s, bl_s, wr_s):
    return pl.pallas_call(
        _tc_layer1_body,
        grid=(NBLK,),
        in_specs=[
            pl.BlockSpec((ROWB, HIDDEN), lambda i: (i, 0)),
            pl.BlockSpec((ROWB, 1), lambda i: (i, 0)),
            pl.BlockSpec((ROWB, HIDDEN), lambda i: (i, 0)),
            pl.BlockSpec((1, 3, HIDDEN, HIDDEN), lambda i: (i // (NBLK // 2), 0, 0, 0)),
            pl.BlockSpec((1, 3, HIDDEN), lambda i: (i // (NBLK // 2), 0, 0)),
            pl.BlockSpec((1, 3, HIDDEN, HIDDEN), lambda i: (i // (NBLK // 2), 0, 0, 0)),
        ],
        out_specs=pl.BlockSpec((ROWB, HIDDEN), lambda i: (i, 0)),
        out_shape=jax.ShapeDtypeStruct((NCONF * N_PAD, HIDDEN), F32),
    )(agg1, w_col, x_all, wl_s, bl_s, wr_s)


def _tc_layer2_body(agg_ref, h1_ref, w_ref, c_ref, wl_ref, bl_ref, wr_ref,
                    es_ref):
    i = pl.program_id(0)
    blocks_per_conf = NBLK // NCONF
    mean = agg_ref[...] * w_ref[...]
    h2 = _dotT(mean, wl_ref[0, 1]) + bl_ref[0, 1][None, :] + _dotT(h1_ref[...], wr_ref[0, 1])
    h2 = jnp.maximum(h2, 0.0)
    s_l = jnp.sum(wl_ref[0, 2], axis=0)[:, None]
    s_r = jnp.sum(wr_ref[0, 2], axis=0)[:, None]
    t = lax.dot_general(h2, s_l, (((1,), (0,)), ((), ())),
                        preferred_element_type=F32,
                        precision=lax.Precision.HIGHEST)
    u = lax.dot_general(h2, s_r, (((1,), (0,)), ((), ())),
                        preferred_element_type=F32,
                        precision=lax.Precision.HIGHEST)
    row0 = (i % blocks_per_conf) * ROWB
    node_id = row0 + lax.broadcasted_iota(jnp.int32, (ROWB, 1), 0)
    valid = node_id < N_BIG
    contrib = jnp.where(valid, c_ref[...] * t + u, 0.0)

    @pl.when(i % blocks_per_conf == 0)
    def _():
        es_ref[...] = (N_BIG * jnp.sum(bl_ref[0, 2])).reshape(1, 1, 1)

    es_ref[...] += jnp.sum(contrib).reshape(1, 1, 1)


def _tc_layer2(agg2, h1, w_col, c_col, wl_s, bl_s, wr_s):
    return pl.pallas_call(
        _tc_layer2_body,
        grid=(NBLK,),
        in_specs=[
            pl.BlockSpec((ROWB, HIDDEN), lambda i: (i, 0)),
            pl.BlockSpec((ROWB, HIDDEN), lambda i: (i, 0)),
            pl.BlockSpec((ROWB, 1), lambda i: (i, 0)),
            pl.BlockSpec((ROWB, 1), lambda i: (i, 0)),
            pl.BlockSpec((1, 3, HIDDEN, HIDDEN), lambda i: (i // (NBLK // 2), 0, 0, 0)),
            pl.BlockSpec((1, 3, HIDDEN), lambda i: (i // (NBLK // 2), 0, 0)),
            pl.BlockSpec((1, 3, HIDDEN, HIDDEN), lambda i: (i // (NBLK // 2), 0, 0, 0)),
        ],
        out_specs=pl.BlockSpec((1, 1, 1), lambda i: (i // (NBLK // NCONF), 0, 0)),
        out_shape=jax.ShapeDtypeStruct((NCONF, 1, 1), F32),
    )(agg2, h1, w_col, c_col, wl_s, bl_s, wr_s)


L_N = 64
L_E = 2048
L_G = 20


def _tc_ligand_body(x_ref, src_ref, dst_ref, wl_ref, bl_ref, wr_ref, out_ref):
    src = src_ref[0, 0, :]
    dst = dst_ref[0, 0, :]
    iota = lax.broadcasted_iota(jnp.int32, (L_E, L_N), 1)
    oh_s = (src[:, None] == iota).astype(F32)
    oh_d = (dst[:, None] == iota).astype(F32)
    A = lax.dot_general(oh_d, oh_s, (((0,), (0,)), ((), ())),
                        preferred_element_type=F32,
                        precision=lax.Precision.HIGHEST)
    denom = jnp.maximum(jnp.sum(A, axis=1, keepdims=True), 1.0)
    h = x_ref[0]
    for i in range(3):
        agg = lax.dot_general(A, h, (((1,), (0,)), ((), ())),
                              preferred_element_type=F32,
                              precision=lax.Precision.HIGHEST)
        h = _dotT(agg / denom, wl_ref[i]) + bl_ref[i][None, :] + _dotT(h, wr_ref[i])
        if i < 2:
            h = jnp.maximum(h, 0.0)
    out_ref[...] = jnp.sum(h).reshape(1, 1, 1)


def _tc_ligand(l_x, l_src, l_dst, l_Wl, l_bl, l_Wr):
    return pl.pallas_call(
        _tc_ligand_body,
        grid=(L_G,),
        in_specs=[
            pl.BlockSpec((1, L_N, HIDDEN), lambda i: (i, 0, 0)),
            pl.BlockSpec((1, 1, L_E), lambda i: (i, 0, 0)),
            pl.BlockSpec((1, 1, L_E), lambda i: (i, 0, 0)),
            pl.BlockSpec((3, HIDDEN, HIDDEN), lambda i: (0, 0, 0)),
            pl.BlockSpec((3, HIDDEN), lambda i: (0, 0)),
            pl.BlockSpec((3, HIDDEN, HIDDEN), lambda i: (0, 0, 0)),
        ],
        out_specs=pl.BlockSpec((1, 1, 1), lambda i: (i, 0, 0)),
        out_shape=jax.ShapeDtypeStruct((L_G, 1, 1), F32),
    )(l_x, l_src, l_dst, l_Wl, l_bl, l_Wr)


def kernel(pl_x, pl_edge_index, pl_edge_attr, p_x, p_edge_index, p_edge_attr,
           l_x, l_edge_index, l_edge_attr,
           pl_Wl, pl_bl, pl_Wr, p_Wl, p_bl, p_Wr, l_Wl, l_bl, l_Wr):
    del pl_edge_attr, p_edge_attr, l_edge_attr
    x_all = jnp.pad(jnp.concatenate([pl_x, p_x]),
                    ((0, 0), (0, N_PAD - N_BIG), (0, 0))).reshape(NCONF * N_PAD, HIDDEN)
    src3d = jnp.concatenate(
        [pl_edge_index[:, 0, :], p_edge_index[:, 0, :]]).astype(jnp.int32).reshape(
            NCONF * NS * NSUPER, KB, CHUNK)
    dst3d = jnp.concatenate(
        [pl_edge_index[:, 1, :], p_edge_index[:, 1, :]]).astype(jnp.int32).reshape(
            NCONF * NS * NSUPER, KB, CHUNK)
    wl_s = jnp.stack([pl_Wl, p_Wl])
    bl_s = jnp.stack([pl_bl, p_bl])
    wr_s = jnp.stack([pl_Wr, p_Wr])

    agg1, w_flat = _sc_aggregate_first(src3d, dst3d, x_all)
    w_col = w_flat.reshape(-1, 1)
    h1 = _tc_layer1(agg1, w_col, x_all, wl_s, bl_s, wr_s)
    agg2, cvec = _sc_aggregate_second(src3d, dst3d, h1, w_flat)
    es = _tc_layer2(agg2, h1, w_col, cvec.reshape(-1, 1), wl_s, bl_s, wr_s)

    l_src = l_edge_index[:, 0:1, :].astype(jnp.int32)
    l_dst = l_edge_index[:, 1:2, :].astype(jnp.int32)
    l_es = _tc_ligand(l_x, l_src, l_dst, l_Wl, l_bl, l_Wr)

    pl_avg = jnp.mean(es[0:2, 0, 0])
    p_avg = jnp.mean(es[2:4, 0, 0])
    l_avg = jnp.mean(l_es[:, 0, 0])
    return (pl_avg - p_avg - l_avg) / (-RT)


# speedup vs baseline: 8.5078x; 1.1019x over previous
import jax
import jax.numpy as jnp
from jax import lax
from jax.experimental import pallas as pl
from jax.experimental.pallas import tpu as pltpu
from jax.experimental.pallas import tpu_sc as plsc

HIDDEN = 128
N_BIG = 10000
N_PAD = 10240
E_BIG = 320000
NCONF = 4
NC = 2
NS = 16
CHUNK = 80
KB = 2
EPT = E_BIG // NS
NSUPER = EPT // (KB * CHUNK)
NPAIR = (NSUPER + 1) // 2
RPT = N_PAD // NS
ZROWS = 16
TEMPERATURE = 298.0
RT = 1.98720425864083 / 1000 * TEMPERATURE
F32 = jnp.float32


def _fill_zrow_zvec(zrow, zvec):
    def zr(i, carry):
        for j in range(HIDDEN // 16):
            zrow[i, pl.ds(j * 16, 16)] = jnp.zeros((16,), F32)
        return carry

    lax.fori_loop(0, zrow.shape[0], zr, 0)

    def zv(i, carry):
        zvec[pl.ds(i * 16, 16)] = jnp.zeros((16,), F32)
        return carry

    lax.fori_loop(0, zvec.shape[0] // 16, zv, 0)


def _zero_slices(s, acc, vec_acc, zrow, zvec):
    rbase = s * RPT
    for t in range(RPT // ZROWS):
        pltpu.sync_copy(zrow, acc.at[pl.ds(rbase + t * ZROWS, ZROWS)])
    pltpu.sync_copy(zvec, vec_acc.at[pl.ds(rbase, RPT)])


def _fire_gathers(tbl, idxb, j0, rows, sem):
    for j in range(KB):
        pltpu.async_copy(tbl.at[idxb.at[j0 + j]], rows.at[j], sem)


def _drain_gathers(tbl, idxb, j0, rows, sem):
    for j in range(KB):
        pltpu.make_async_copy(tbl.at[idxb.at[j0 + j]], rows.at[j], sem).wait()


def _fire_scatters(rows, acc, idxb, j0, sem):
    for j in range(KB):
        pltpu.async_copy(rows.at[j], acc.at[idxb.at[j0 + j]], sem, add=True)


def _drain_scatters(rows, acc, idxb, j0, sem):
    for j in range(KB):
        pltpu.make_async_copy(rows.at[j], acc.at[idxb.at[j0 + j]], sem).wait()


def _fire_vec_scatters(vals, vacc, idxb, j0, sem):
    for j in range(KB):
        pltpu.async_copy(vals.at[j], vacc.at[idxb.at[j0 + j]], sem, add=True)


def _drain_vec_scatters(vals, vacc, idxb, j0, sem):
    for j in range(KB):
        pltpu.make_async_copy(vals.at[j], vacc.at[idxb.at[j0 + j]], sem).wait()


def _sc_first_body(idxb_hbm, x_hbm, agg_hbm, w_hbm,
                   ib0, ib1,
                   rows0, rows1, ones2, wtmp, zrow, zvec,
                   acc, dacc, g0s, g1s, s0s, s1s, o0s, o1s):
    c = lax.axis_index("c")
    s = lax.axis_index("s")
    _fill_zrow_zvec(zrow, zvec)
    for j in range(KB):
        for i in range(CHUNK // 16):
            ones2[j, pl.ds(i * 16, 16)] = jnp.ones((16,), F32)
    for cg in range(NCONF // NC):
        conf = c + NC * cg
        noff = conf * N_PAD
        _zero_slices(s, acc, dacc, zrow, zvec)
        plsc.subcore_barrier()
        base3 = (conf * NS + s) * NSUPER

        pltpu.sync_copy(idxb_hbm.at[base3], ib0)
        _fire_gathers(x_hbm, ib0, 0, rows0, g0s)

        def it(i, carry):
            sc1 = 2 * i + 1
            sc2 = 2 * i + 2
            _drain_gathers(x_hbm, ib0, 0, rows0, g0s)
            _fire_scatters(rows0, acc, ib0, KB, s0s)
            _fire_vec_scatters(ones2, dacc, ib0, KB, o0s)

            @pl.when(i > 0)
            def _():
                _drain_scatters(rows1, acc, ib1, KB, s1s)
                _drain_vec_scatters(ones2, dacc, ib1, KB, o1s)

            @pl.when(sc1 < NSUPER)
            def _():
                pltpu.sync_copy(idxb_hbm.at[base3 + sc1], ib1)
                _fire_gathers(x_hbm, ib1, 0, rows1, g1s)
                _drain_gathers(x_hbm, ib1, 0, rows1, g1s)
                _fire_scatters(rows1, acc, ib1, KB, s1s)
                _fire_vec_scatters(ones2, dacc, ib1, KB, o1s)

            _drain_scatters(rows0, acc, ib0, KB, s0s)
            _drain_vec_scatters(ones2, dacc, ib0, KB, o0s)

            @pl.when(sc2 < NSUPER)
            def _():
                pltpu.sync_copy(idxb_hbm.at[base3 + sc2], ib0)
                _fire_gathers(x_hbm, ib0, 0, rows0, g0s)

            return carry

        lax.fori_loop(0, NPAIR, it, 0)
        plsc.subcore_barrier()

        rb = s * RPT
        pltpu.sync_copy(dacc.at[pl.ds(rb, RPT)], wtmp)

        def winv(i, carry):
            sl = pl.ds(i * 16, 16)
            wtmp[sl] = 1.0 / jnp.maximum(wtmp[sl], 1.0)
            return carry

        lax.fori_loop(0, RPT // 16, winv, 0)
        pltpu.sync_copy(wtmp, w_hbm.at[pl.ds(noff + rb, RPT)])
        pltpu.sync_copy(acc.at[pl.ds(rb, RPT)], agg_hbm.at[pl.ds(noff + rb, RPT)])
        plsc.subcore_barrier()


def _sc_second_body(idxb_hbm, h_hbm, w_hbm, agg_hbm, c_hbm,
                    ib0, ib1,
                    rows0, rows1, wv0, wv1, zrow, zvec,
                    acc, cacc, g0s, g1s, s0s, s1s, o0s, o1s):
    c = lax.axis_index("c")
    s = lax.axis_index("s")
    _fill_zrow_zvec(zrow, zvec)
    for cg in range(NCONF // NC):
        conf = c + NC * cg
        noff = conf * N_PAD
        _zero_slices(s, acc, cacc, zrow, zvec)
        plsc.subcore_barrier()
        base3 = (conf * NS + s) * NSUPER

        pltpu.sync_copy(idxb_hbm.at[base3], ib0)
        _fire_gathers(h_hbm, ib0, 0, rows0, g0s)
        _fire_gathers(w_hbm, ib0, 2 * KB, wv0, g0s)

        def it(i, carry):
            sc1 = 2 * i + 1
            sc2 = 2 * i + 2
            _drain_gathers(h_hbm, ib0, 0, rows0, g0s)
            _drain_gathers(w_hbm, ib0, 2 * KB, wv0, g0s)
            _fire_scatters(rows0, acc, ib0, KB, s0s)
            _fire_vec_scatters(wv0, cacc, ib0, 3 * KB, o0s)

            @pl.when(i > 0)
            def _():
                _drain_scatters(rows1, acc, ib1, KB, s1s)
                _drain_vec_scatters(wv1, cacc, ib1, 3 * KB, o1s)

            @pl.when(sc1 < NSUPER)
            def _():
                pltpu.sync_copy(idxb_hbm.at[base3 + sc1], ib1)
                _fire_gathers(h_hbm, ib1, 0, rows1, g1s)
                _fire_gathers(w_hbm, ib1, 2 * KB, wv1, g1s)
                _drain_gathers(h_hbm, ib1, 0, rows1, g1s)
                _drain_gathers(w_hbm, ib1, 2 * KB, wv1, g1s)
                _fire_scatters(rows1, acc, ib1, KB, s1s)
                _fire_vec_scatters(wv1, cacc, ib1, 3 * KB, o1s)

            _drain_scatters(rows0, acc, ib0, KB, s0s)
            _drain_vec_scatters(wv0, cacc, ib0, 3 * KB, o0s)

            @pl.when(sc2 < NSUPER)
            def _():
                pltpu.sync_copy(idxb_hbm.at[base3 + sc2], ib0)
                _fire_gathers(h_hbm, ib0, 0, rows0, g0s)
                _fire_gathers(w_hbm, ib0, 2 * KB, wv0, g0s)

            return carry

        lax.fori_loop(0, NPAIR, it, 0)
        plsc.subcore_barrier()
        rb = s * RPT
        pltpu.sync_copy(acc.at[pl.ds(rb, RPT)], agg_hbm.at[pl.ds(noff + rb, RPT)])
        pltpu.sync_copy(cacc.at[pl.ds(rb, RPT)], c_hbm.at[pl.ds(noff + rb, RPT)])
        plsc.subcore_barrier()


def _sc_aggregate_first(idxb, x_all):
    mesh = plsc.VectorSubcoreMesh(core_axis_name="c", subcore_axis_name="s")
    return pl.kernel(
        _sc_first_body,
        mesh=mesh,
        out_type=[
            jax.ShapeDtypeStruct((NCONF * N_PAD, HIDDEN), F32),
            jax.ShapeDtypeStruct((NCONF * N_PAD,), F32),
        ],
        scratch_types=[
            pltpu.VMEM((2 * KB, CHUNK), jnp.int32),
            pltpu.VMEM((2 * KB, CHUNK), jnp.int32),
            pltpu.VMEM((KB, CHUNK, HIDDEN), F32),
            pltpu.VMEM((KB, CHUNK, HIDDEN), F32),
            pltpu.VMEM((KB, CHUNK), F32),
            pltpu.VMEM((RPT,), F32),
            pltpu.VMEM((ZROWS, HIDDEN), F32),
            pltpu.VMEM((RPT,), F32),
            pltpu.VMEM_SHARED((N_PAD, HIDDEN), F32),
            pltpu.VMEM_SHARED((N_PAD,), F32),
            pltpu.SemaphoreType.DMA,
            pltpu.SemaphoreType.DMA,
            pltpu.SemaphoreType.DMA,
            pltpu.SemaphoreType.DMA,
            pltpu.SemaphoreType.DMA,
            pltpu.SemaphoreType.DMA,
        ],
    )(idxb, x_all)


def _sc_aggregate_second(idxb, h_all, w_flat):
    mesh = plsc.VectorSubcoreMesh(core_axis_name="c", subcore_axis_name="s")
    return pl.kernel(
        _sc_second_body,
        mesh=mesh,
        out_type=[
            jax.ShapeDtypeStruct((NCONF * N_PAD, HIDDEN), F32),
            jax.ShapeDtypeStruct((NCONF * N_PAD,), F32),
        ],
        scratch_types=[
            pltpu.VMEM((4 * KB, CHUNK), jnp.int32),
            pltpu.VMEM((4 * KB, CHUNK), jnp.int32),
            pltpu.VMEM((KB, CHUNK, HIDDEN), F32),
            pltpu.VMEM((KB, CHUNK, HIDDEN), F32),
            pltpu.VMEM((KB, CHUNK), F32),
            pltpu.VMEM((KB, CHUNK), F32),
            pltpu.VMEM((ZROWS, HIDDEN), F32),
            pltpu.VMEM((RPT,), F32),
            pltpu.VMEM_SHARED((N_PAD, HIDDEN), F32),
            pltpu.VMEM_SHARED((N_PAD,), F32),
            pltpu.SemaphoreType.DMA,
            pltpu.SemaphoreType.DMA,
            pltpu.SemaphoreType.DMA,
            pltpu.SemaphoreType.DMA,
            pltpu.SemaphoreType.DMA,
            pltpu.SemaphoreType.DMA,
        ],
    )(idxb, h_all, w_flat)


ROWB = 2048
NBLK = NCONF * N_PAD // ROWB


def _dotT(a, b):
    return lax.dot_general(a, b, (((1,), (1,)), ((), ())),
                           preferred_element_type=F32,
                           precision=lax.Precision.HIGHEST)


def _tc_layer1_body(agg_ref, w_ref, x_ref, wl_ref, bl_ref, wr_ref, h1_ref):
    mean = agg_ref[...] * w_ref[...]
    h = _dotT(mean, wl_ref[0, 0]) + bl_ref[0, 0][None, :] + _dotT(x_ref[...], wr_ref[0, 0])
    h1_ref[...] = jnp.maximum(h, 0.0)


def _tc_layer1(agg1, w_col, x_all, wl_s, bl_s, wr_s):
    return pl.pallas_call(
        _tc_layer1_body,
        grid=(NBLK,),
        in_specs=[
            pl.BlockSpec((ROWB, HIDDEN), lambda i: (i, 0)),
            pl.BlockSpec((ROWB, 1), lambda i: (i, 0)),
            pl.BlockSpec((ROWB, HIDDEN), lambda i: (i, 0)),
            pl.BlockSpec((1, 3, HIDDEN, HIDDEN), lambda i: (i // (NBLK // 2), 0, 0, 0)),
            pl.BlockSpec((1, 3, HIDDEN), lambda i: (i // (NBLK // 2), 0, 0)),
            pl.BlockSpec((1, 3, HIDDEN, HIDDEN), lambda i: (i // (NBLK // 2), 0, 0, 0)),
        ],
        out_specs=pl.BlockSpec((ROWB, HIDDEN), lambda i: (i, 0)),
        out_shape=jax.ShapeDtypeStruct((NCONF * N_PAD, HIDDEN), F32),
    )(agg1, w_col, x_all, wl_s, bl_s, wr_s)


def _tc_layer2_body(agg_ref, h1_ref, w_ref, c_ref, wl_ref, bl_ref, wr_ref,
                    es_ref):
    i = pl.program_id(0)
    blocks_per_conf = NBLK // NCONF
    mean = agg_ref[...] * w_ref[...]
    h2 = _dotT(mean, wl_ref[0, 1]) + bl_ref[0, 1][None, :] + _dotT(h1_ref[...], wr_ref[0, 1])
    h2 = jnp.maximum(h2, 0.0)
    s_l = jnp.sum(wl_ref[0, 2], axis=0)[:, None]
    s_r = jnp.sum(wr_ref[0, 2], axis=0)[:, None]
    t = lax.dot_general(h2, s_l, (((1,), (0,)), ((), ())),
                        preferred_element_type=F32,
                        precision=lax.Precision.HIGHEST)
    u = lax.dot_general(h2, s_r, (((1,), (0,)), ((), ())),
                        preferred_element_type=F32,
                        precision=lax.Precision.HIGHEST)
    row0 = (i % blocks_per_conf) * ROWB
    node_id = row0 + lax.broadcasted_iota(jnp.int32, (ROWB, 1), 0)
    valid = node_id < N_BIG
    contrib = jnp.where(valid, c_ref[...] * t + u, 0.0)

    @pl.when(i % blocks_per_conf == 0)
    def _():
        es_ref[...] = (N_BIG * jnp.sum(bl_ref[0, 2])).reshape(1, 1, 1)

    es_ref[...] += jnp.sum(contrib).reshape(1, 1, 1)


def _tc_layer2(agg2, h1, w_col, c_col, wl_s, bl_s, wr_s):
    return pl.pallas_call(
        _tc_layer2_body,
        grid=(NBLK,),
        in_specs=[
            pl.BlockSpec((ROWB, HIDDEN), lambda i: (i, 0)),
            pl.BlockSpec((ROWB, HIDDEN), lambda i: (i, 0)),
            pl.BlockSpec((ROWB, 1), lambda i: (i, 0)),
            pl.BlockSpec((ROWB, 1), lambda i: (i, 0)),
            pl.BlockSpec((1, 3, HIDDEN, HIDDEN), lambda i: (i // (NBLK // 2), 0, 0, 0)),
            pl.BlockSpec((1, 3, HIDDEN), lambda i: (i // (NBLK // 2), 0, 0)),
            pl.BlockSpec((1, 3, HIDDEN, HIDDEN), lambda i: (i // (NBLK // 2), 0, 0, 0)),
        ],
        out_specs=pl.BlockSpec((1, 1, 1), lambda i: (i // (NBLK // NCONF), 0, 0)),
        out_shape=jax.ShapeDtypeStruct((NCONF, 1, 1), F32),
    )(agg2, h1, w_col, c_col, wl_s, bl_s, wr_s)


L_N = 64
L_E = 2048
L_G = 20


def _tc_ligand_body(x_ref, src_ref, dst_ref, wl_ref, bl_ref, wr_ref, out_ref):
    src = src_ref[0, 0, :]
    dst = dst_ref[0, 0, :]
    iota = lax.broadcasted_iota(jnp.int32, (L_E, L_N), 1)
    oh_s = (src[:, None] == iota).astype(F32)
    oh_d = (dst[:, None] == iota).astype(F32)
    A = lax.dot_general(oh_d, oh_s, (((0,), (0,)), ((), ())),
                        preferred_element_type=F32,
                        precision=lax.Precision.HIGHEST)
    denom = jnp.maximum(jnp.sum(A, axis=1, keepdims=True), 1.0)
    h = x_ref[0]
    for i in range(3):
        agg = lax.dot_general(A, h, (((1,), (0,)), ((), ())),
                              preferred_element_type=F32,
                              precision=lax.Precision.HIGHEST)
        h = _dotT(agg / denom, wl_ref[i]) + bl_ref[i][None, :] + _dotT(h, wr_ref[i])
        if i < 2:
            h = jnp.maximum(h, 0.0)
    out_ref[...] = jnp.sum(h).reshape(1, 1, 1)


def _tc_ligand(l_x, l_src, l_dst, l_Wl, l_bl, l_Wr):
    return pl.pallas_call(
        _tc_ligand_body,
        grid=(L_G,),
        in_specs=[
            pl.BlockSpec((1, L_N, HIDDEN), lambda i: (i, 0, 0)),
            pl.BlockSpec((1, 1, L_E), lambda i: (i, 0, 0)),
            pl.BlockSpec((1, 1, L_E), lambda i: (i, 0, 0)),
            pl.BlockSpec((3, HIDDEN, HIDDEN), lambda i: (0, 0, 0)),
            pl.BlockSpec((3, HIDDEN), lambda i: (0, 0)),
            pl.BlockSpec((3, HIDDEN, HIDDEN), lambda i: (0, 0, 0)),
        ],
        out_specs=pl.BlockSpec((1, 1, 1), lambda i: (i, 0, 0)),
        out_shape=jax.ShapeDtypeStruct((L_G, 1, 1), F32),
    )(l_x, l_src, l_dst, l_Wl, l_bl, l_Wr)


def kernel(pl_x, pl_edge_index, pl_edge_attr, p_x, p_edge_index, p_edge_attr,
           l_x, l_edge_index, l_edge_attr,
           pl_Wl, pl_bl, pl_Wr, p_Wl, p_bl, p_Wr, l_Wl, l_bl, l_Wr):
    del pl_edge_attr, p_edge_attr, l_edge_attr
    x_all = jnp.pad(jnp.concatenate([pl_x, p_x]),
                    ((0, 0), (0, N_PAD - N_BIG), (0, 0))).reshape(NCONF * N_PAD, HIDDEN)
    nrows = NCONF * NS * NSUPER
    conf_off = (jnp.arange(NCONF, dtype=jnp.int32) * N_PAD)[:, None]
    srci = jnp.concatenate(
        [pl_edge_index[:, 0, :], p_edge_index[:, 0, :]]).astype(jnp.int32)
    dsti = jnp.concatenate(
        [pl_edge_index[:, 1, :], p_edge_index[:, 1, :]]).astype(jnp.int32)
    src_adj = (srci + conf_off).reshape(nrows, KB, CHUNK)
    dst_raw = dsti.reshape(nrows, KB, CHUNK)
    dst_adj = (dsti + conf_off).reshape(nrows, KB, CHUNK)
    src_raw = srci.reshape(nrows, KB, CHUNK)
    idxb_a = jnp.concatenate([src_adj, dst_raw], axis=1)
    idxb_c = jnp.concatenate([src_adj, dst_raw, dst_adj, src_raw], axis=1)
    dst3d = jnp.concatenate(
        [pl_edge_index[:, 1, :], p_edge_index[:, 1, :]]).astype(jnp.int32).reshape(
            NCONF * NS * NSUPER, KB, CHUNK)
    wl_s = jnp.stack([pl_Wl, p_Wl])
    bl_s = jnp.stack([pl_bl, p_bl])
    wr_s = jnp.stack([pl_Wr, p_Wr])

    agg1, w_flat = _sc_aggregate_first(idxb_a, x_all)
    w_col = w_flat.reshape(-1, 1)
    h1 = _tc_layer1(agg1, w_col, x_all, wl_s, bl_s, wr_s)
    agg2, cvec = _sc_aggregate_second(idxb_c, h1, w_flat)
    es = _tc_layer2(agg2, h1, w_col, cvec.reshape(-1, 1), wl_s, bl_s, wr_s)

    l_src = l_edge_index[:, 0:1, :].astype(jnp.int32)
    l_dst = l_edge_index[:, 1:2, :].astype(jnp.int32)
    l_es = _tc_ligand(l_x, l_src, l_dst, l_Wl, l_bl, l_Wr)

    pl_avg = jnp.mean(es[0:2, 0, 0])
    p_avg = jnp.mean(es[2:4, 0, 0])
    l_avg = jnp.mean(l_es[:, 0, 0])
    return (pl_avg - p_avg - l_avg) / (-RT)
